# baseline (speedup 1.0000x reference)
import jax
import jax.numpy as jnp
from jax import lax
from jax.experimental import pallas as pl
from jax.experimental.pallas import tpu as pltpu
from jax.experimental.pallas import tpu_sc as plsc

NN = 10000
NP = 10240
DD = 128
NCLS = 40
E_REAL = 320000 + NN

NCORES = 2
NSUB = 16
NW = NCORES * NSUB
LANES = 16
SUB = 64
NSC0 = 196
NSC1 = 128
EP = (NSC0 + NSC1) * SUB * NSUB + 2 * SUB

ROWS_TC = 1024
GRID_TC = NP // ROWS_TC
ROWS_PER_TILE = NP // NSUB


def _dense_body(x_ref, w_ref, as_ref, ad_ref,
                h_ref, asrc_ref, adst_ref, cmax_ref, msum_ref):
    i = pl.program_id(0)
    h = jnp.dot(x_ref[...], w_ref[...], preferred_element_type=jnp.float32)
    h_ref[...] = h
    a = jnp.dot(h, as_ref[...], preferred_element_type=jnp.float32)
    d = jnp.dot(h, ad_ref[...], preferred_element_type=jnp.float32)
    asrc_ref[...] = a
    adst_ref[...] = d

    @pl.when(i == 0)
    def _():
        msum_ref[0] = -jnp.inf
        msum_ref[1] = -jnp.inf

    msum_ref[0] = jnp.maximum(msum_ref[0], jnp.max(a))
    msum_ref[1] = jnp.maximum(msum_ref[1], jnp.max(d))

    @pl.when(i == pl.num_programs(0) - 1)
    def _():
        cmax_ref[...] = jnp.full(
            (1, LANES), jnp.maximum(msum_ref[0] + msum_ref[1], 0.0),
            jnp.float32)


def _dense_call(x, w, a_s, a_d):
    return pl.pallas_call(
        _dense_body,
        grid=(GRID_TC,),
        in_specs=[
            pl.BlockSpec((ROWS_TC, DD), lambda i: (i, 0)),
            pl.BlockSpec((DD, DD), lambda i: (0, 0)),
            pl.BlockSpec((DD, 1), lambda i: (0, 0)),
            pl.BlockSpec((DD, 1), lambda i: (0, 0)),
        ],
        out_specs=[
            pl.BlockSpec((ROWS_TC, DD), lambda i: (i, 0)),
            pl.BlockSpec((ROWS_TC, 1), lambda i: (i, 0)),
            pl.BlockSpec((ROWS_TC, 1), lambda i: (i, 0)),
            pl.BlockSpec((1, LANES), lambda i: (0, 0)),
        ],
        out_shape=[
            jax.ShapeDtypeStruct((NP, DD), jnp.float32),
            jax.ShapeDtypeStruct((NP, 1), jnp.float32),
            jax.ShapeDtypeStruct((NP, 1), jnp.float32),
            jax.ShapeDtypeStruct((1, LANES), jnp.float32),
        ],
        scratch_shapes=[pltpu.SMEM((2,), jnp.float32)],
    )(x, w, a_s, a_d)


def _mid_body(p_ref, dp_ref, b_ref, w_ref, as_ref, ad_ref,
              h_ref, asrc_ref, adst_ref, cmax_ref, msum_ref):
    i = pl.program_id(0)
    den = jnp.sum(dp_ref[...], axis=0) + 1e-16
    agg = (p_ref[0] + p_ref[1]) / den + b_ref[...]
    hin = jnp.maximum(agg, 0.0)
    h = jnp.dot(hin, w_ref[...], preferred_element_type=jnp.float32)
    h_ref[...] = h
    a = jnp.dot(h, as_ref[...], preferred_element_type=jnp.float32)
    d = jnp.dot(h, ad_ref[...], preferred_element_type=jnp.float32)
    asrc_ref[...] = a
    adst_ref[...] = d

    @pl.when(i == 0)
    def _():
        msum_ref[0] = -jnp.inf
        msum_ref[1] = -jnp.inf

    msum_ref[0] = jnp.maximum(msum_ref[0], jnp.max(a))
    msum_ref[1] = jnp.maximum(msum_ref[1], jnp.max(d))

    @pl.when(i == pl.num_programs(0) - 1)
    def _():
        cmax_ref[...] = jnp.full(
            (1, LANES), jnp.maximum(msum_ref[0] + msum_ref[1], 0.0),
            jnp.float32)


def _mid_call(p, dp, b, w, a_s, a_d):
    return pl.pallas_call(
        _mid_body,
        grid=(GRID_TC,),
        in_specs=[
            pl.BlockSpec((2, ROWS_TC, DD), lambda i: (0, i, 0)),
            pl.BlockSpec((NW, ROWS_TC, 1), lambda i: (0, i, 0)),
            pl.BlockSpec((1, DD), lambda i: (0, 0)),
            pl.BlockSpec((DD, DD), lambda i: (0, 0)),
            pl.BlockSpec((DD, 1), lambda i: (0, 0)),
            pl.BlockSpec((DD, 1), lambda i: (0, 0)),
        ],
        out_specs=[
            pl.BlockSpec((ROWS_TC, DD), lambda i: (i, 0)),
            pl.BlockSpec((ROWS_TC, 1), lambda i: (i, 0)),
            pl.BlockSpec((ROWS_TC, 1), lambda i: (i, 0)),
            pl.BlockSpec((1, LANES), lambda i: (0, 0)),
        ],
        out_shape=[
            jax.ShapeDtypeStruct((NP, DD), jnp.float32),
            jax.ShapeDtypeStruct((NP, 1), jnp.float32),
            jax.ShapeDtypeStruct((NP, 1), jnp.float32),
            jax.ShapeDtypeStruct((1, LANES), jnp.float32),
        ],
        scratch_shapes=[pltpu.SMEM((2,), jnp.float32)],
    )(p, dp, b, w, a_s, a_d)


def _fin_body(p_ref, dp_ref, b_ref, wc_ref, bc_ref, o_ref, acc_ref):
    i = pl.program_id(0)
    den = jnp.sum(dp_ref[...], axis=0) + 1e-16
    agg = jnp.maximum((p_ref[0] + p_ref[1]) / den + b_ref[...], 0.0)
    gid = i * ROWS_TC + lax.broadcasted_iota(jnp.int32, (ROWS_TC, 1), 0)
    agg = jnp.where(gid < NN, agg, 0.0)
    s = jnp.sum(agg, axis=0, keepdims=True)

    @pl.when(i == 0)
    def _():
        acc_ref[...] = jnp.zeros_like(acc_ref)

    acc_ref[...] += s

    @pl.when(i == pl.num_programs(0) - 1)
    def _():
        g = acc_ref[...] * (1.0 / NN)
        o_ref[...] = jnp.dot(g, wc_ref[...],
                             preferred_element_type=jnp.float32) + bc_ref[...]


def _fin_call(p, dp, b, wc, bc):
    return pl.pallas_call(
        _fin_body,
        grid=(GRID_TC,),
        in_specs=[
            pl.BlockSpec((2, ROWS_TC, DD), lambda i: (0, i, 0)),
            pl.BlockSpec((NW, ROWS_TC, 1), lambda i: (0, i, 0)),
            pl.BlockSpec((1, DD), lambda i: (0, 0)),
            pl.BlockSpec((DD, NCLS), lambda i: (0, 0)),
            pl.BlockSpec((1, NCLS), lambda i: (0, 0)),
        ],
        out_specs=pl.BlockSpec((1, NCLS), lambda i: (0, 0)),
        out_shape=jax.ShapeDtypeStruct((1, NCLS), jnp.float32),
        scratch_shapes=[pltpu.VMEM((1, DD), jnp.float32)],
    )(p, dp, b, wc, bc)


def _sc_edge_body(src_hbm, dst_hbm, asrc_hbm, adst_hbm, cvec_hbm, h_hbm,
                  out_hbm, den_hbm,
                  asrc_v, adst_v, cvec_v, den_v,
                  ex0, ex1, si0, si1, di0, di1, sg0, sg1, dg0, dg1,
                  rows0, rows1, shared_out, g0, g1, s0, s1, i0, i1):
    exb = (ex0, ex1)
    sib = (si0, si1)
    dib = (di0, di1)
    sgat = (sg0, sg1)
    dsct = (dg0, dg1)
    rowsb = (rows0, rows1)
    gsem = (g0, g1)
    ssem = (s0, s1)
    isem = (i0, i1)

    cid = lax.axis_index("c")
    sid = lax.axis_index("s")
    wid = cid * NSUB + sid
    nsc = jnp.where(cid == 0, NSC0, NSC1)
    per_w = nsc * SUB
    base = cid * (NSUB * SUB * NSC0) + sid * per_w
    rbase = sid * ROWS_PER_TILE

    pltpu.sync_copy(asrc_hbm, asrc_v)
    pltpu.sync_copy(adst_hbm, adst_v)
    pltpu.sync_copy(cvec_hbm.at[0], cvec_v)

    zero16 = jnp.zeros((LANES,), jnp.float32)

    def _zden(i, c):
        den_v[pl.ds(i * LANES, LANES)] = zero16
        return c
    lax.fori_loop(0, NP // LANES, _zden, 0)

    def _zrow(r, c):
        for k in range(DD // LANES):
            rows0[r, pl.ds(k * LANES, LANES)] = zero16
        return c
    lax.fori_loop(0, SUB, _zrow, 0)
    for i in range(ROWS_PER_TILE // SUB):
        pltpu.sync_copy(rows0,
                        shared_out.at[pl.ds(rbase + i * SUB, SUB)])
    plsc.subcore_barrier()

    cvec = cvec_v[...]

    def ex_compute(c, b):
        off = c * SUB
        for k in range(SUB // LANES):
            s16 = sib[b][pl.ds(k * LANES, LANES)]
            d16 = dib[b][pl.ds(k * LANES, LANES)]
            av = plsc.load_gather(asrc_v, [s16])
            dv = plsc.load_gather(adst_v, [d16])
            e = av + dv
            e = jnp.where(e >= 0.0, e, e * 0.2)
            ex = jnp.exp(e - cvec)
            gid = base + off + k * LANES + lax.iota(jnp.int32, LANES)
            ex = jnp.where(gid < E_REAL, ex, 0.0)
            plsc.addupdate_scatter(den_v, [d16], ex)
            exb[b][pl.ds(k * LANES, LANES)] = ex

    def copy_idx_for_streams(b):
        for k in range(SUB // LANES):
            sl = pl.ds(k * LANES, LANES)
            sgat[b][sl] = sib[b][sl]
            dsct[b][sl] = dib[b][sl]

    def issue_idx(c, b):
        off = c * SUB
        pltpu.async_copy(src_hbm.at[pl.ds(base + off, SUB)], sib[b], isem[b])
        pltpu.async_copy(dst_hbm.at[pl.ds(base + off, SUB)], dib[b], isem[b])

    def wait_idx(b):
        pltpu.make_async_copy(src_hbm.at[pl.ds(0, SUB)], sib[b],
                              isem[b]).wait()
        pltpu.make_async_copy(dst_hbm.at[pl.ds(0, SUB)], dib[b],
                              isem[b]).wait()

    def issue_gather(b):
        pltpu.async_copy(h_hbm.at[sgat[b]], rowsb[b], gsem[b])

    def wait_gather(b):
        pltpu.make_async_copy(h_hbm.at[sgat[b]], rowsb[b], gsem[b]).wait()

    def issue_scatter(b):
        pltpu.async_copy(rowsb[b], shared_out.at[dsct[b]], ssem[b], add=True)

    def wait_scatter(b):
        pltpu.make_async_copy(rowsb[b], shared_out.at[dsct[b]],
                              ssem[b]).wait()

    def scale(b):
        def _s(g, cc):
            w16 = exb[b][pl.ds(g * LANES, LANES)]
            rb = g * LANES
            for l in range(LANES):
                w = w16[l]
                for k in range(DD // LANES):
                    rowsb[b][rb + l, pl.ds(k * LANES, LANES)] = (
                        rowsb[b][rb + l, pl.ds(k * LANES, LANES)] * w)
            return cc
        lax.fori_loop(0, SUB // LANES, _s, 0)

    for b in range(2):
        off = b * SUB
        pltpu.sync_copy(src_hbm.at[pl.ds(base + off, SUB)], sib[b])
        pltpu.sync_copy(dst_hbm.at[pl.ds(base + off, SUB)], dib[b])
        ex_compute(b, b)
        copy_idx_for_streams(b)
        issue_idx(b + 2, b)
        issue_gather(b)

    def steady(c2, carry):
        c = 2 * c2
        for b in range(2):
            cb = c + b
            wait_gather(b)
            scale(b)
            issue_scatter(b)
            wait_idx(b)
            ex_compute(cb + 2, b)
            wait_scatter(b)
            copy_idx_for_streams(b)
            issue_idx(cb + 4, b)
            issue_gather(b)
        return carry
    lax.fori_loop(0, nsc // 2 - 1, steady, 0)

    for b in range(2):
        wait_gather(b)
        scale(b)
        issue_scatter(b)
        wait_idx(b)
    for b in range(2):
        wait_scatter(b)

    plsc.subcore_barrier()

    pltpu.sync_copy(den_v, den_hbm.at[wid])
    pltpu.sync_copy(shared_out.at[pl.ds(rbase, ROWS_PER_TILE)],
                    out_hbm.at[cid, pl.ds(rbase, ROWS_PER_TILE)])


_sc_edge = pl.kernel(
    _sc_edge_body,
    out_type=(
        jax.ShapeDtypeStruct((NCORES, NP, DD), jnp.float32),
        jax.ShapeDtypeStruct((NW, NP), jnp.float32),
    ),
    mesh=plsc.VectorSubcoreMesh(core_axis_name="c", subcore_axis_name="s"),
    compiler_params=pltpu.CompilerParams(needs_layout_passes=False),
    scratch_types=[
        pltpu.VMEM((NP,), jnp.float32),
        pltpu.VMEM((NP,), jnp.float32),
        pltpu.VMEM((LANES,), jnp.float32),
        pltpu.VMEM((NP,), jnp.float32),
        pltpu.VMEM((SUB,), jnp.float32),
        pltpu.VMEM((SUB,), jnp.float32),
        pltpu.VMEM((SUB,), jnp.int32),
        pltpu.VMEM((SUB,), jnp.int32),
        pltpu.VMEM((SUB,), jnp.int32),
        pltpu.VMEM((SUB,), jnp.int32),
        pltpu.VMEM((SUB,), jnp.int32),
        pltpu.VMEM((SUB,), jnp.int32),
        pltpu.VMEM((SUB,), jnp.int32),
        pltpu.VMEM((SUB,), jnp.int32),
        pltpu.VMEM((SUB, DD), jnp.float32),
        pltpu.VMEM((SUB, DD), jnp.float32),
        pltpu.VMEM_SHARED((NP, DD), jnp.float32),
        pltpu.SemaphoreType.DMA,
        pltpu.SemaphoreType.DMA,
        pltpu.SemaphoreType.DMA,
        pltpu.SemaphoreType.DMA,
        pltpu.SemaphoreType.DMA,
        pltpu.SemaphoreType.DMA,
    ],
)


def kernel(x, edge_index, W1, a_src1, a_dst1, b1, W2, a_src2, a_dst2, b2,
           Wc, bc):
    loop = jnp.arange(NN, dtype=jnp.int32)
    pad = jnp.zeros((EP - E_REAL,), jnp.int32)
    src = jnp.concatenate([edge_index[0].astype(jnp.int32), loop, pad])
    dst = jnp.concatenate([edge_index[1].astype(jnp.int32), loop, pad])
    xp = jnp.concatenate(
        [x, jnp.zeros((NP - NN, DD), jnp.float32)], axis=0)

    h1, asrc1, adst1, cmax1 = _dense_call(
        xp, W1, a_src1.reshape(DD, 1), a_dst1.reshape(DD, 1))
    p1, d1 = _sc_edge(src, dst, asrc1.reshape(NP), adst1.reshape(NP),
                      cmax1, h1)

    h2, asrc2, adst2, cmax2 = _mid_call(
        p1, d1.reshape(NW, NP, 1), b1.reshape(1, DD), W2,
        a_src2.reshape(DD, 1), a_dst2.reshape(DD, 1))
    p2, d2 = _sc_edge(src, dst, asrc2.reshape(NP), adst2.reshape(NP),
                      cmax2, h2)

    return _fin_call(p2, d2.reshape(NW, NP, 1), b2.reshape(1, DD), Wc,
                     bc.reshape(1, NCLS))


# speedup vs baseline: 1.4358x; 1.4358x over previous
import jax
import jax.numpy as jnp
from jax import lax
from jax.experimental import pallas as pl
from jax.experimental.pallas import tpu as pltpu
from jax.experimental.pallas import tpu_sc as plsc

NN = 10000
NP = 10240
DD = 128
NCLS = 40
E_REAL = 320000 + NN

NCORES = 2
NSUB = 16
NW = NCORES * NSUB
LANES = 16
SUB = 64
NSC0 = 196
NSC1 = 128
EP = (NSC0 + NSC1) * SUB * NSUB + 2 * SUB

ROWS_TC = 1024
GRID_TC = NP // ROWS_TC
ROWS_PER_TILE = NP // NSUB


def _dense_body(x_ref, w_ref, as_ref, ad_ref,
                h_ref, asrc_ref, adst_ref, cmax_ref, msum_ref):
    i = pl.program_id(0)
    h = jnp.dot(x_ref[...], w_ref[...], preferred_element_type=jnp.float32)
    h_ref[...] = h
    a = jnp.dot(h, as_ref[...], preferred_element_type=jnp.float32)
    d = jnp.dot(h, ad_ref[...], preferred_element_type=jnp.float32)
    asrc_ref[...] = a
    adst_ref[...] = d

    @pl.when(i == 0)
    def _():
        msum_ref[0] = -jnp.inf
        msum_ref[1] = -jnp.inf

    msum_ref[0] = jnp.maximum(msum_ref[0], jnp.max(a))
    msum_ref[1] = jnp.maximum(msum_ref[1], jnp.max(d))

    @pl.when(i == pl.num_programs(0) - 1)
    def _():
        cmax_ref[...] = jnp.full(
            (1, 1), jnp.maximum(msum_ref[0] + msum_ref[1], 0.0), jnp.float32)


def _dense_call(x, w, a_s, a_d):
    return pl.pallas_call(
        _dense_body,
        grid=(GRID_TC,),
        in_specs=[
            pl.BlockSpec((ROWS_TC, DD), lambda i: (i, 0)),
            pl.BlockSpec((DD, DD), lambda i: (0, 0)),
            pl.BlockSpec((DD, 1), lambda i: (0, 0)),
            pl.BlockSpec((DD, 1), lambda i: (0, 0)),
        ],
        out_specs=[
            pl.BlockSpec((ROWS_TC, DD), lambda i: (i, 0)),
            pl.BlockSpec((ROWS_TC, 1), lambda i: (i, 0)),
            pl.BlockSpec((ROWS_TC, 1), lambda i: (i, 0)),
            pl.BlockSpec((1, 1), lambda i: (0, 0)),
        ],
        out_shape=[
            jax.ShapeDtypeStruct((NP, DD), jnp.float32),
            jax.ShapeDtypeStruct((NP, 1), jnp.float32),
            jax.ShapeDtypeStruct((NP, 1), jnp.float32),
            jax.ShapeDtypeStruct((1, 1), jnp.float32),
        ],
        scratch_shapes=[pltpu.SMEM((2,), jnp.float32)],
    )(x, w, a_s, a_d)


def _mid_body(p_ref, dp_ref, b_ref, w_ref, as_ref, ad_ref,
              h_ref, asrc_ref, adst_ref, cmax_ref, msum_ref):
    i = pl.program_id(0)
    den = dp_ref[...] + 1e-16
    agg = (p_ref[0] + p_ref[1]) / den + b_ref[...]
    hin = jnp.maximum(agg, 0.0)
    h = jnp.dot(hin, w_ref[...], preferred_element_type=jnp.float32)
    h_ref[...] = h
    a = jnp.dot(h, as_ref[...], preferred_element_type=jnp.float32)
    d = jnp.dot(h, ad_ref[...], preferred_element_type=jnp.float32)
    asrc_ref[...] = a
    adst_ref[...] = d

    @pl.when(i == 0)
    def _():
        msum_ref[0] = -jnp.inf
        msum_ref[1] = -jnp.inf

    msum_ref[0] = jnp.maximum(msum_ref[0], jnp.max(a))
    msum_ref[1] = jnp.maximum(msum_ref[1], jnp.max(d))

    @pl.when(i == pl.num_programs(0) - 1)
    def _():
        cmax_ref[...] = jnp.full(
            (1, 1), jnp.maximum(msum_ref[0] + msum_ref[1], 0.0), jnp.float32)


def _mid_call(p, dp, b, w, a_s, a_d):
    return pl.pallas_call(
        _mid_body,
        grid=(GRID_TC,),
        in_specs=[
            pl.BlockSpec((2, ROWS_TC, DD), lambda i: (0, i, 0)),
            pl.BlockSpec((ROWS_TC, 1), lambda i: (i, 0)),
            pl.BlockSpec((1, DD), lambda i: (0, 0)),
            pl.BlockSpec((DD, DD), lambda i: (0, 0)),
            pl.BlockSpec((DD, 1), lambda i: (0, 0)),
            pl.BlockSpec((DD, 1), lambda i: (0, 0)),
        ],
        out_specs=[
            pl.BlockSpec((ROWS_TC, DD), lambda i: (i, 0)),
            pl.BlockSpec((ROWS_TC, 1), lambda i: (i, 0)),
            pl.BlockSpec((ROWS_TC, 1), lambda i: (i, 0)),
            pl.BlockSpec((1, 1), lambda i: (0, 0)),
        ],
        out_shape=[
            jax.ShapeDtypeStruct((NP, DD), jnp.float32),
            jax.ShapeDtypeStruct((NP, 1), jnp.float32),
            jax.ShapeDtypeStruct((NP, 1), jnp.float32),
            jax.ShapeDtypeStruct((1, 1), jnp.float32),
        ],
        scratch_shapes=[pltpu.SMEM((2,), jnp.float32)],
    )(p, dp, b, w, a_s, a_d)


def _fin_body(p_ref, dp_ref, b_ref, wc_ref, bc_ref, o_ref, acc_ref):
    i = pl.program_id(0)
    den = dp_ref[...] + 1e-16
    agg = jnp.maximum((p_ref[0] + p_ref[1]) / den + b_ref[...], 0.0)
    gid = i * ROWS_TC + lax.broadcasted_iota(jnp.int32, (ROWS_TC, 1), 0)
    agg = jnp.where(gid < NN, agg, 0.0)
    s = jnp.sum(agg, axis=0, keepdims=True)

    @pl.when(i == 0)
    def _():
        acc_ref[...] = jnp.zeros_like(acc_ref)

    acc_ref[...] += s

    @pl.when(i == pl.num_programs(0) - 1)
    def _():
        g = acc_ref[...] * (1.0 / NN)
        o_ref[...] = jnp.dot(g, wc_ref[...],
                             preferred_element_type=jnp.float32) + bc_ref[...]


def _fin_call(p, dp, b, wc, bc):
    return pl.pallas_call(
        _fin_body,
        grid=(GRID_TC,),
        in_specs=[
            pl.BlockSpec((2, ROWS_TC, DD), lambda i: (0, i, 0)),
            pl.BlockSpec((ROWS_TC, 1), lambda i: (i, 0)),
            pl.BlockSpec((1, DD), lambda i: (0, 0)),
            pl.BlockSpec((DD, NCLS), lambda i: (0, 0)),
            pl.BlockSpec((1, NCLS), lambda i: (0, 0)),
        ],
        out_specs=pl.BlockSpec((1, NCLS), lambda i: (0, 0)),
        out_shape=jax.ShapeDtypeStruct((1, NCLS), jnp.float32),
        scratch_shapes=[pltpu.VMEM((1, DD), jnp.float32)],
    )(p, dp, b, wc, bc)


def _densum_body(dp_ref, o_ref):
    o_ref[...] = jnp.sum(dp_ref[...], axis=0).reshape(NP, 1)


def _densum_call(dp):
    return pl.pallas_call(
        _densum_body,
        out_shape=jax.ShapeDtypeStruct((NP, 1), jnp.float32),
    )(dp)


def _sc_edge_body(src_hbm, dst_hbm, asrc_hbm, adst_hbm, cvec_hbm, h_hbm,
                  out_hbm, den_hbm,
                  asrc_v, adst_v, cvec_v, den_v,
                  ex0, ex1, si0, si1, di0, di1, sg0, sg1, dg0, dg1,
                  rows0, rows1, shared_out, g0, g1, s0, s1, i0, i1):
    exb = (ex0, ex1)
    sib = (si0, si1)
    dib = (di0, di1)
    sgat = (sg0, sg1)
    dsct = (dg0, dg1)
    rowsb = (rows0, rows1)
    gsem = (g0, g1)
    ssem = (s0, s1)
    isem = (i0, i1)

    cid = lax.axis_index("c")
    sid = lax.axis_index("s")
    wid = cid * NSUB + sid
    nsc = jnp.where(cid == 0, NSC0, NSC1)
    per_w = nsc * SUB
    base = cid * (NSUB * SUB * NSC0) + sid * per_w
    rbase = sid * ROWS_PER_TILE

    pltpu.sync_copy(asrc_hbm, asrc_v)
    pltpu.sync_copy(adst_hbm, adst_v)
    pltpu.sync_copy(cvec_hbm, cvec_v)

    zero16 = jnp.zeros((LANES,), jnp.float32)

    def _zden(i, c):
        den_v[pl.ds(i * LANES, LANES)] = zero16
        return c
    lax.fori_loop(0, NP // LANES, _zden, 0)

    def _zrow(r, c):
        for k in range(DD // LANES):
            rows0[r, pl.ds(k * LANES, LANES)] = zero16
        return c
    lax.fori_loop(0, SUB, _zrow, 0)
    for i in range(ROWS_PER_TILE // SUB):
        pltpu.sync_copy(rows0,
                        shared_out.at[pl.ds(rbase + i * SUB, SUB)])
    plsc.subcore_barrier()

    cvec = cvec_v[...]

    def ex_compute(c, b):
        off = c * SUB
        for k in range(SUB // LANES):
            s16 = sib[b][pl.ds(k * LANES, LANES)]
            d16 = dib[b][pl.ds(k * LANES, LANES)]
            av = plsc.load_gather(asrc_v, [s16])
            dv = plsc.load_gather(adst_v, [d16])
            e = av + dv
            e = jnp.where(e >= 0.0, e, e * 0.2)
            ex = jnp.exp(e - cvec)
            gid = base + off + k * LANES + lax.iota(jnp.int32, LANES)
            ex = jnp.where(gid < E_REAL, ex, 0.0)
            plsc.addupdate_scatter(den_v, [d16], ex)
            exb[b][pl.ds(k * LANES, LANES)] = ex

    def copy_idx_for_streams(b):
        for k in range(SUB // LANES):
            sl = pl.ds(k * LANES, LANES)
            sgat[b][sl] = sib[b][sl]
            dsct[b][sl] = dib[b][sl]

    def issue_idx(c, b):
        off = c * SUB
        pltpu.async_copy(src_hbm.at[pl.ds(base + off, SUB)], sib[b], isem[b])
        pltpu.async_copy(dst_hbm.at[pl.ds(base + off, SUB)], dib[b], isem[b])

    def wait_idx(b):
        pltpu.make_async_copy(src_hbm.at[pl.ds(0, SUB)], sib[b],
                              isem[b]).wait()
        pltpu.make_async_copy(dst_hbm.at[pl.ds(0, SUB)], dib[b],
                              isem[b]).wait()

    def issue_gather(b):
        pltpu.async_copy(h_hbm.at[sgat[b]], rowsb[b], gsem[b])

    def wait_gather(b):
        pltpu.make_async_copy(h_hbm.at[sgat[b]], rowsb[b], gsem[b]).wait()

    def issue_scatter(b):
        pltpu.async_copy(rowsb[b], shared_out.at[dsct[b]], ssem[b], add=True)

    def wait_scatter(b):
        pltpu.make_async_copy(rowsb[b], shared_out.at[dsct[b]],
                              ssem[b]).wait()

    def scale(b):
        def _s(g, cc):
            w16 = exb[b][pl.ds(g * LANES, LANES)]
            rb = g * LANES
            for l in range(LANES):
                w = w16[l]
                for k in range(DD // LANES):
                    rowsb[b][rb + l, pl.ds(k * LANES, LANES)] = (
                        rowsb[b][rb + l, pl.ds(k * LANES, LANES)] * w)
            return cc
        lax.fori_loop(0, SUB // LANES, _s, 0)

    for b in range(2):
        off = b * SUB
        pltpu.sync_copy(src_hbm.at[pl.ds(base + off, SUB)], sib[b])
        pltpu.sync_copy(dst_hbm.at[pl.ds(base + off, SUB)], dib[b])
        ex_compute(b, b)
        copy_idx_for_streams(b)
        issue_idx(b + 2, b)
        issue_gather(b)

    def steady(c2, carry):
        c = 2 * c2
        for b in range(2):
            cb = c + b
            wait_gather(b)
            scale(b)
            issue_scatter(b)
            wait_idx(b)
            ex_compute(cb + 2, b)
            wait_scatter(b)
            copy_idx_for_streams(b)
            issue_idx(cb + 4, b)
            issue_gather(b)
        return carry
    lax.fori_loop(0, nsc // 2 - 1, steady, 0)

    for b in range(2):
        wait_gather(b)
        scale(b)
        issue_scatter(b)
        wait_idx(b)
    for b in range(2):
        wait_scatter(b)

    plsc.subcore_barrier()

    pltpu.sync_copy(den_v, den_hbm.at[wid])
    pltpu.sync_copy(shared_out.at[pl.ds(rbase, ROWS_PER_TILE)],
                    out_hbm.at[cid, pl.ds(rbase, ROWS_PER_TILE)])


_sc_edge = pl.kernel(
    _sc_edge_body,
    out_type=(
        jax.ShapeDtypeStruct((NCORES, NP, DD), jnp.float32),
        jax.ShapeDtypeStruct((NW, NP), jnp.float32),
    ),
    mesh=plsc.VectorSubcoreMesh(core_axis_name="c", subcore_axis_name="s"),
    compiler_params=pltpu.CompilerParams(needs_layout_passes=False),
    scratch_types=[
        pltpu.VMEM((NP,), jnp.float32),
        pltpu.VMEM((NP,), jnp.float32),
        pltpu.VMEM((LANES,), jnp.float32),
        pltpu.VMEM((NP,), jnp.float32),
        pltpu.VMEM((SUB,), jnp.float32),
        pltpu.VMEM((SUB,), jnp.float32),
        pltpu.VMEM((SUB,), jnp.int32),
        pltpu.VMEM((SUB,), jnp.int32),
        pltpu.VMEM((SUB,), jnp.int32),
        pltpu.VMEM((SUB,), jnp.int32),
        pltpu.VMEM((SUB,), jnp.int32),
        pltpu.VMEM((SUB,), jnp.int32),
        pltpu.VMEM((SUB,), jnp.int32),
        pltpu.VMEM((SUB,), jnp.int32),
        pltpu.VMEM((SUB, DD), jnp.float32),
        pltpu.VMEM((SUB, DD), jnp.float32),
        pltpu.VMEM_SHARED((NP, DD), jnp.float32),
        pltpu.SemaphoreType.DMA,
        pltpu.SemaphoreType.DMA,
        pltpu.SemaphoreType.DMA,
        pltpu.SemaphoreType.DMA,
        pltpu.SemaphoreType.DMA,
        pltpu.SemaphoreType.DMA,
    ],
)


def kernel(x, edge_index, W1, a_src1, a_dst1, b1, W2, a_src2, a_dst2, b2,
           Wc, bc):
    loop = jnp.arange(NN, dtype=jnp.int32)
    pad = jnp.zeros((EP - E_REAL,), jnp.int32)
    src = jnp.concatenate([edge_index[0].astype(jnp.int32), loop, pad])
    dst = jnp.concatenate([edge_index[1].astype(jnp.int32), loop, pad])
    xp = jnp.concatenate(
        [x, jnp.zeros((NP - NN, DD), jnp.float32)], axis=0)

    h1, asrc1, adst1, cmax1 = _dense_call(
        xp, W1, a_src1.reshape(DD, 1), a_dst1.reshape(DD, 1))
    cvec1 = jnp.broadcast_to(cmax1.reshape(1), (LANES,))
    p1, d1 = _sc_edge(src, dst, asrc1.reshape(NP), adst1.reshape(NP),
                      cvec1, h1)

    h2, asrc2, adst2, cmax2 = _mid_call(
        p1, _densum_call(d1), b1.reshape(1, DD), W2,
        a_src2.reshape(DD, 1), a_dst2.reshape(DD, 1))
    cvec2 = jnp.broadcast_to(cmax2.reshape(1), (LANES,))
    p2, d2 = _sc_edge(src, dst, asrc2.reshape(NP), adst2.reshape(NP),
                      cvec2, h2)

    return _fin_call(p2, _densum_call(d2), b2.reshape(1, DD), Wc,
                     bc.reshape(1, NCLS))


# speedup vs baseline: 1.4642x; 1.0197x over previous
import jax
import jax.numpy as jnp
from jax import lax
from jax.experimental import pallas as pl
from jax.experimental.pallas import tpu as pltpu
from jax.experimental.pallas import tpu_sc as plsc

NN = 10000
NP = 10240
DD = 128
NCLS = 40
E_REAL = 320000 + NN

NCORES = 2
NSUB = 16
NW = NCORES * NSUB
LANES = 16
SUB = 64
NSC0 = 190
NSC1 = 134
EP = (NSC0 + NSC1) * SUB * NSUB + 2 * SUB

ROWS_TC = 1024
GRID_TC = NP // ROWS_TC
ROWS_PER_TILE = NP // NSUB


def _dense_body(x_ref, w_ref, as_ref, ad_ref,
                h_ref, asrc_ref, adst_ref, cmax_ref, msum_ref):
    i = pl.program_id(0)
    h = jnp.dot(x_ref[...], w_ref[...], preferred_element_type=jnp.float32)
    h_ref[...] = h
    a = jnp.dot(h, as_ref[...], preferred_element_type=jnp.float32)
    d = jnp.dot(h, ad_ref[...], preferred_element_type=jnp.float32)
    asrc_ref[...] = a
    adst_ref[...] = d

    @pl.when(i == 0)
    def _():
        msum_ref[0] = -jnp.inf
        msum_ref[1] = -jnp.inf

    msum_ref[0] = jnp.maximum(msum_ref[0], jnp.max(a))
    msum_ref[1] = jnp.maximum(msum_ref[1], jnp.max(d))

    @pl.when(i == pl.num_programs(0) - 1)
    def _():
        cmax_ref[...] = jnp.full(
            (1, 1), jnp.maximum(msum_ref[0] + msum_ref[1], 0.0), jnp.float32)


def _dense_call(x, w, a_s, a_d):
    return pl.pallas_call(
        _dense_body,
        grid=(GRID_TC,),
        in_specs=[
            pl.BlockSpec((ROWS_TC, DD), lambda i: (i, 0)),
            pl.BlockSpec((DD, DD), lambda i: (0, 0)),
            pl.BlockSpec((DD, 1), lambda i: (0, 0)),
            pl.BlockSpec((DD, 1), lambda i: (0, 0)),
        ],
        out_specs=[
            pl.BlockSpec((ROWS_TC, DD), lambda i: (i, 0)),
            pl.BlockSpec((ROWS_TC, 1), lambda i: (i, 0)),
            pl.BlockSpec((ROWS_TC, 1), lambda i: (i, 0)),
            pl.BlockSpec((1, 1), lambda i: (0, 0)),
        ],
        out_shape=[
            jax.ShapeDtypeStruct((NP, DD), jnp.float32),
            jax.ShapeDtypeStruct((NP, 1), jnp.float32),
            jax.ShapeDtypeStruct((NP, 1), jnp.float32),
            jax.ShapeDtypeStruct((1, 1), jnp.float32),
        ],
        scratch_shapes=[pltpu.SMEM((2,), jnp.float32)],
    )(x, w, a_s, a_d)


def _mid_body(p_ref, dp_ref, b_ref, w_ref, as_ref, ad_ref,
              h_ref, asrc_ref, adst_ref, cmax_ref, msum_ref):
    i = pl.program_id(0)
    den = dp_ref[...] + 1e-16
    agg = (p_ref[0] + p_ref[1]) / den + b_ref[...]
    hin = jnp.maximum(agg, 0.0)
    h = jnp.dot(hin, w_ref[...], preferred_element_type=jnp.float32)
    h_ref[...] = h
    a = jnp.dot(h, as_ref[...], preferred_element_type=jnp.float32)
    d = jnp.dot(h, ad_ref[...], preferred_element_type=jnp.float32)
    asrc_ref[...] = a
    adst_ref[...] = d

    @pl.when(i == 0)
    def _():
        msum_ref[0] = -jnp.inf
        msum_ref[1] = -jnp.inf

    msum_ref[0] = jnp.maximum(msum_ref[0], jnp.max(a))
    msum_ref[1] = jnp.maximum(msum_ref[1], jnp.max(d))

    @pl.when(i == pl.num_programs(0) - 1)
    def _():
        cmax_ref[...] = jnp.full(
            (1, 1), jnp.maximum(msum_ref[0] + msum_ref[1], 0.0), jnp.float32)


def _mid_call(p, dp, b, w, a_s, a_d):
    return pl.pallas_call(
        _mid_body,
        grid=(GRID_TC,),
        in_specs=[
            pl.BlockSpec((2, ROWS_TC, DD), lambda i: (0, i, 0)),
            pl.BlockSpec((ROWS_TC, 1), lambda i: (i, 0)),
            pl.BlockSpec((1, DD), lambda i: (0, 0)),
            pl.BlockSpec((DD, DD), lambda i: (0, 0)),
            pl.BlockSpec((DD, 1), lambda i: (0, 0)),
            pl.BlockSpec((DD, 1), lambda i: (0, 0)),
        ],
        out_specs=[
            pl.BlockSpec((ROWS_TC, DD), lambda i: (i, 0)),
            pl.BlockSpec((ROWS_TC, 1), lambda i: (i, 0)),
            pl.BlockSpec((ROWS_TC, 1), lambda i: (i, 0)),
            pl.BlockSpec((1, 1), lambda i: (0, 0)),
        ],
        out_shape=[
            jax.ShapeDtypeStruct((NP, DD), jnp.float32),
            jax.ShapeDtypeStruct((NP, 1), jnp.float32),
            jax.ShapeDtypeStruct((NP, 1), jnp.float32),
            jax.ShapeDtypeStruct((1, 1), jnp.float32),
        ],
        scratch_shapes=[pltpu.SMEM((2,), jnp.float32)],
    )(p, dp, b, w, a_s, a_d)


def _fin_body(p_ref, dp_ref, b_ref, wc_ref, bc_ref, o_ref, acc_ref):
    i = pl.program_id(0)
    den = dp_ref[...] + 1e-16
    agg = jnp.maximum((p_ref[0] + p_ref[1]) / den + b_ref[...], 0.0)
    gid = i * ROWS_TC + lax.broadcasted_iota(jnp.int32, (ROWS_TC, 1), 0)
    agg = jnp.where(gid < NN, agg, 0.0)
    s = jnp.sum(agg, axis=0, keepdims=True)

    @pl.when(i == 0)
    def _():
        acc_ref[...] = jnp.zeros_like(acc_ref)

    acc_ref[...] += s

    @pl.when(i == pl.num_programs(0) - 1)
    def _():
        g = acc_ref[...] * (1.0 / NN)
        o_ref[...] = jnp.dot(g, wc_ref[...],
                             preferred_element_type=jnp.float32) + bc_ref[...]


def _fin_call(p, dp, b, wc, bc):
    return pl.pallas_call(
        _fin_body,
        grid=(GRID_TC,),
        in_specs=[
            pl.BlockSpec((2, ROWS_TC, DD), lambda i: (0, i, 0)),
            pl.BlockSpec((ROWS_TC, 1), lambda i: (i, 0)),
            pl.BlockSpec((1, DD), lambda i: (0, 0)),
            pl.BlockSpec((DD, NCLS), lambda i: (0, 0)),
            pl.BlockSpec((1, NCLS), lambda i: (0, 0)),
        ],
        out_specs=pl.BlockSpec((1, NCLS), lambda i: (0, 0)),
        out_shape=jax.ShapeDtypeStruct((1, NCLS), jnp.float32),
        scratch_shapes=[pltpu.VMEM((1, DD), jnp.float32)],
    )(p, dp, b, wc, bc)


def _densum_body(dp_ref, o_ref):
    o_ref[...] = jnp.sum(dp_ref[...], axis=0).reshape(NP, 1)


def _densum_call(dp):
    return pl.pallas_call(
        _densum_body,
        out_shape=jax.ShapeDtypeStruct((NP, 1), jnp.float32),
    )(dp)


def _sc_edge_body(src_hbm, dst_hbm, asrc_hbm, adst_hbm, cvec_hbm, h_hbm,
                  out_hbm, den_hbm,
                  asrc_v, adst_v, cvec_v, den_v,
                  ex0, ex1, si0, si1, di0, di1, sg0, sg1, dg0, dg1,
                  rows0, rows1, shared_out, g0, g1, s0, s1, i0, i1):
    exb = (ex0, ex1)
    sib = (si0, si1)
    dib = (di0, di1)
    sgat = (sg0, sg1)
    dsct = (dg0, dg1)
    rowsb = (rows0, rows1)
    gsem = (g0, g1)
    ssem = (s0, s1)
    isem = (i0, i1)

    cid = lax.axis_index("c")
    sid = lax.axis_index("s")
    wid = cid * NSUB + sid
    nsc = jnp.where(cid == 0, NSC0, NSC1)
    per_w = nsc * SUB
    base = cid * (NSUB * SUB * NSC0) + sid * per_w
    rbase = sid * ROWS_PER_TILE

    pltpu.sync_copy(asrc_hbm, asrc_v)
    pltpu.sync_copy(adst_hbm, adst_v)
    pltpu.sync_copy(cvec_hbm, cvec_v)

    zero16 = jnp.zeros((LANES,), jnp.float32)

    def _zden(i, c):
        den_v[pl.ds(i * LANES, LANES)] = zero16
        return c
    lax.fori_loop(0, NP // LANES, _zden, 0)

    def _zrow(r, c):
        for k in range(DD // LANES):
            rows0[r, pl.ds(k * LANES, LANES)] = zero16
        return c
    lax.fori_loop(0, SUB, _zrow, 0)
    for i in range(ROWS_PER_TILE // SUB):
        pltpu.sync_copy(rows0,
                        shared_out.at[pl.ds(rbase + i * SUB, SUB)])
    plsc.subcore_barrier()

    cvec = cvec_v[...]

    def ex_compute(c, b):
        off = c * SUB
        for k in range(SUB // LANES):
            s16 = sib[b][pl.ds(k * LANES, LANES)]
            d16 = dib[b][pl.ds(k * LANES, LANES)]
            av = plsc.load_gather(asrc_v, [s16])
            dv = plsc.load_gather(adst_v, [d16])
            e = av + dv
            e = jnp.where(e >= 0.0, e, e * 0.2)
            ex = jnp.exp(e - cvec)
            gid = base + off + k * LANES + lax.iota(jnp.int32, LANES)
            ex = jnp.where(gid < E_REAL, ex, 0.0)
            plsc.addupdate_scatter(den_v, [d16], ex)
            exb[b][pl.ds(k * LANES, LANES)] = ex

    def copy_idx_for_streams(b):
        for k in range(SUB // LANES):
            sl = pl.ds(k * LANES, LANES)
            sgat[b][sl] = sib[b][sl]
            dsct[b][sl] = dib[b][sl]

    def issue_idx(c, b):
        off = c * SUB
        pltpu.async_copy(src_hbm.at[pl.ds(base + off, SUB)], sib[b], isem[b])
        pltpu.async_copy(dst_hbm.at[pl.ds(base + off, SUB)], dib[b], isem[b])

    def wait_idx(b):
        pltpu.make_async_copy(src_hbm.at[pl.ds(0, SUB)], sib[b],
                              isem[b]).wait()
        pltpu.make_async_copy(dst_hbm.at[pl.ds(0, SUB)], dib[b],
                              isem[b]).wait()

    def issue_gather(b):
        pltpu.async_copy(h_hbm.at[sgat[b]], rowsb[b], gsem[b])

    def wait_gather(b):
        pltpu.make_async_copy(h_hbm.at[sgat[b]], rowsb[b], gsem[b]).wait()

    def issue_scatter(b):
        pltpu.async_copy(rowsb[b], shared_out.at[dsct[b]], ssem[b], add=True)

    def wait_scatter(b):
        pltpu.make_async_copy(rowsb[b], shared_out.at[dsct[b]],
                              ssem[b]).wait()

    def scale(b):
        def _s(g, cc):
            w16 = exb[b][pl.ds(g * LANES, LANES)]
            rb = g * LANES
            for l in range(LANES):
                w = w16[l]
                for k in range(DD // LANES):
                    rowsb[b][rb + l, pl.ds(k * LANES, LANES)] = (
                        rowsb[b][rb + l, pl.ds(k * LANES, LANES)] * w)
            return cc
        lax.fori_loop(0, SUB // LANES, _s, 0)

    for b in range(2):
        off = b * SUB
        pltpu.sync_copy(src_hbm.at[pl.ds(base + off, SUB)], sib[b])
        pltpu.sync_copy(dst_hbm.at[pl.ds(base + off, SUB)], dib[b])
        ex_compute(b, b)
        copy_idx_for_streams(b)
        issue_idx(b + 2, b)
        issue_gather(b)

    def steady(c2, carry):
        c = 2 * c2
        for b in range(2):
            cb = c + b
            wait_gather(b)
            scale(b)
            issue_scatter(b)
            wait_idx(b)
            ex_compute(cb + 2, b)
            wait_scatter(b)
            copy_idx_for_streams(b)
            issue_idx(cb + 4, b)
            issue_gather(b)
        return carry
    lax.fori_loop(0, nsc // 2 - 1, steady, 0)

    for b in range(2):
        wait_gather(b)
        scale(b)
        issue_scatter(b)
        wait_idx(b)
    for b in range(2):
        wait_scatter(b)

    plsc.subcore_barrier()

    pltpu.sync_copy(den_v, den_hbm.at[wid])
    pltpu.sync_copy(shared_out.at[pl.ds(rbase, ROWS_PER_TILE)],
                    out_hbm.at[cid, pl.ds(rbase, ROWS_PER_TILE)])


_sc_edge = pl.kernel(
    _sc_edge_body,
    out_type=(
        jax.ShapeDtypeStruct((NCORES, NP, DD), jnp.float32),
        jax.ShapeDtypeStruct((NW, NP), jnp.float32),
    ),
    mesh=plsc.VectorSubcoreMesh(core_axis_name="c", subcore_axis_name="s"),
    compiler_params=pltpu.CompilerParams(needs_layout_passes=False),
    scratch_types=[
        pltpu.VMEM((NP,), jnp.float32),
        pltpu.VMEM((NP,), jnp.float32),
        pltpu.VMEM((LANES,), jnp.float32),
        pltpu.VMEM((NP,), jnp.float32),
        pltpu.VMEM((SUB,), jnp.float32),
        pltpu.VMEM((SUB,), jnp.float32),
        pltpu.VMEM((SUB,), jnp.int32),
        pltpu.VMEM((SUB,), jnp.int32),
        pltpu.VMEM((SUB,), jnp.int32),
        pltpu.VMEM((SUB,), jnp.int32),
        pltpu.VMEM((SUB,), jnp.int32),
        pltpu.VMEM((SUB,), jnp.int32),
        pltpu.VMEM((SUB,), jnp.int32),
        pltpu.VMEM((SUB,), jnp.int32),
        pltpu.VMEM((SUB, DD), jnp.float32),
        pltpu.VMEM((SUB, DD), jnp.float32),
        pltpu.VMEM_SHARED((NP, DD), jnp.float32),
        pltpu.SemaphoreType.DMA,
        pltpu.SemaphoreType.DMA,
        pltpu.SemaphoreType.DMA,
        pltpu.SemaphoreType.DMA,
        pltpu.SemaphoreType.DMA,
        pltpu.SemaphoreType.DMA,
    ],
)


def kernel(x, edge_index, W1, a_src1, a_dst1, b1, W2, a_src2, a_dst2, b2,
           Wc, bc):
    loop = jnp.arange(NN, dtype=jnp.int32)
    pad = jnp.zeros((EP - E_REAL,), jnp.int32)
    src = jnp.concatenate([edge_index[0].astype(jnp.int32), loop, pad])
    dst = jnp.concatenate([edge_index[1].astype(jnp.int32), loop, pad])
    xp = jnp.concatenate(
        [x, jnp.zeros((NP - NN, DD), jnp.float32)], axis=0)

    h1, asrc1, adst1, cmax1 = _dense_call(
        xp, W1, a_src1.reshape(DD, 1), a_dst1.reshape(DD, 1))
    cvec1 = jnp.broadcast_to(cmax1.reshape(1), (LANES,))
    p1, d1 = _sc_edge(src, dst, asrc1.reshape(NP), adst1.reshape(NP),
                      cvec1, h1)

    h2, asrc2, adst2, cmax2 = _mid_call(
        p1, _densum_call(d1), b1.reshape(1, DD), W2,
        a_src2.reshape(DD, 1), a_dst2.reshape(DD, 1))
    cvec2 = jnp.broadcast_to(cmax2.reshape(1), (LANES,))
    p2, d2 = _sc_edge(src, dst, asrc2.reshape(NP), adst2.reshape(NP),
                      cvec2, h2)

    return _fin_call(p2, _densum_call(d2), b2.reshape(1, DD), Wc,
                     bc.reshape(1, NCLS))
